# Initial kernel scaffold; baseline (speedup 1.0000x reference)
import jax
import jax.numpy as jnp
from jax.experimental import pallas as pl


def kernel(test_feature, train_features, train_labels):
    raise NotImplementedError("write your pallas kernel here")


# speedup vs baseline: 7.1917x; 7.1917x over previous
import functools

import jax
import jax.numpy as jnp
from jax import lax
from jax.experimental import pallas as pl
from jax.experimental.pallas import tpu as pltpu
from jax.experimental.pallas import tpu_sc as plsc

K = 20
NUM_CLASSES = 21
PATCH = 16
IMG = 224
NROWS = 14
D = 384
N_TRAIN = 25088
NQ = 392
NCH = 196
G = 14
CPG = 14
NQP = 400
BIG = 1 << 20


def _matmul_body(q_ref, t_ref, o_ref):
    o_ref[...] = jnp.dot(q_ref[...], t_ref[...],
                         preferred_element_type=jnp.float32)


def _similarity(q, train_features, nblk=3584):
    return pl.pallas_call(
        _matmul_body,
        grid=(N_TRAIN // nblk,),
        in_specs=[
            pl.BlockSpec((NQ, D), lambda i: (0, 0)),
            pl.BlockSpec((D, nblk), lambda i: (0, i)),
        ],
        out_specs=pl.BlockSpec((NQ, nblk), lambda i: (0, i)),
        out_shape=jax.ShapeDtypeStruct((NQ, N_TRAIN), jnp.float32),
    )(q, train_features)


def _topk_body(sim_ref, idx_ref, w_ref, scr_ref):
    lane = lax.broadcasted_iota(jnp.int32, (8, 128), 1)
    neg = jnp.full((8, 128), -jnp.inf, jnp.float32)
    zero_i = jnp.zeros((8, 128), jnp.int32)

    M1, M2, M3, J1, J2, J3 = [], [], [], [], [], []
    for g in range(G):
        m1 = m2 = m3 = neg
        j1 = j2 = j3 = zero_i
        for cc in range(CPG):
            c = g * CPG + cc
            x = sim_ref[:, c, :]
            jc = jnp.full((8, 128), c, jnp.int32)
            gt1 = x > m1
            gt2 = x > m2
            gt3 = x > m3
            nm1 = jnp.where(gt1, x, m1)
            nm2 = jnp.where(gt1, m1, jnp.where(gt2, x, m2))
            nm3 = jnp.where(gt2, m2, jnp.where(gt3, x, m3))
            nj1 = jnp.where(gt1, jc, j1)
            nj2 = jnp.where(gt1, j1, jnp.where(gt2, jc, j2))
            nj3 = jnp.where(gt2, j2, jnp.where(gt3, jc, j3))
            m1, m2, m3, j1, j2, j3 = nm1, nm2, nm3, nj1, nj2, nj3
        M1.append(m1); M2.append(m2); M3.append(m3)
        J1.append(j1); J2.append(j2); J3.append(j3)

    kiota = lax.broadcasted_iota(jnp.int32, (8, K), 1)
    vals20 = jnp.zeros((8, K), jnp.float32)
    idxs20 = jnp.zeros((8, K), jnp.int32)
    flag = jnp.zeros((8, 1), jnp.bool_)
    neg1 = jnp.full((8, 1), -jnp.inf, jnp.float32)
    for k in range(K):
        mall = M1[0]
        for g in range(1, G):
            mall = jnp.maximum(mall, M1[g])
        v = jnp.max(mall, axis=1, keepdims=True)
        done = jnp.zeros((8, 1), jnp.bool_)
        jwin = jnp.zeros((8, 1), jnp.int32)
        lwin = jnp.zeros((8, 1), jnp.int32)
        for g in range(G):
            eq = M1[g] == v
            hit = jnp.any(eq, axis=1, keepdims=True)
            take = hit & (~done)
            done = done | hit
            sel = eq & take
            lidx = jnp.min(jnp.where(sel, lane, BIG), axis=1, keepdims=True)
            W = sel & (lane == lidx)
            flag = flag | jnp.any(W & (M2[g] == -jnp.inf),
                                  axis=1, keepdims=True)
            jwin = jwin + jnp.sum(jnp.where(W, J1[g], 0), axis=1,
                                  keepdims=True)
            lwin = lwin + jnp.sum(jnp.where(W, lane, 0), axis=1,
                                  keepdims=True)
            M1[g] = jnp.where(W, M2[g], M1[g])
            M2[g] = jnp.where(W, M3[g], M2[g])
            M3[g] = jnp.where(W, neg, M3[g])
            J1[g] = jnp.where(W, J2[g], J1[g])
            J2[g] = jnp.where(W, J3[g], J2[g])
        sel_k = kiota == k
        vals20 = jnp.where(sel_k, jnp.broadcast_to(v, (8, K)), vals20)
        idxs20 = jnp.where(sel_k, jnp.broadcast_to(jwin * 128 + lwin, (8, K)),
                           idxs20)

    def _emit(vals, idxs):
        m = jnp.max(vals, axis=1, keepdims=True)
        e = jnp.exp(vals - m)
        w = e / jnp.sum(e, axis=1, keepdims=True)
        idx_ref[...] = idxs
        w_ref[...] = w

    _emit(vals20, idxs20)

    @pl.when(jnp.any(flag))
    def _redo():
        for c in range(NCH):
            scr_ref[:, c, :] = sim_ref[:, c, :]
        lane3 = lax.broadcasted_iota(jnp.int32, (8, 1, 128), 2)

        def round_body(k, carry):
            vals, idxs = carry

            def find_body(c, fc):
                bv, bi = fc
                x = scr_ref[:, pl.ds(c, 1), :]
                cm = jnp.max(x, axis=(1, 2), keepdims=True)[:, :, 0]
                eq = x == cm[:, :, None]
                li = jnp.min(jnp.where(eq, lane3, BIG), axis=(1, 2),
                             keepdims=True)[:, :, 0]
                upd = cm > bv
                bi = jnp.where(upd, c * 128 + li, bi)
                bv = jnp.where(upd, cm, bv)
                return bv, bi

            bv, bi = lax.fori_loop(
                0, NCH, find_body,
                (jnp.full((8, 1), -jnp.inf, jnp.float32),
                 jnp.zeros((8, 1), jnp.int32)))

            def rm_body(c, _):
                x = scr_ref[:, pl.ds(c, 1), :]
                gi = c * 128 + lane3
                scr_ref[:, pl.ds(c, 1), :] = jnp.where(
                    gi == bi[:, :, None], -jnp.inf, x)
                return 0

            lax.fori_loop(0, NCH, rm_body, 0)
            sk = kiota == k
            vals = jnp.where(sk, jnp.broadcast_to(bv, (8, K)), vals)
            idxs = jnp.where(sk, jnp.broadcast_to(bi, (8, K)), idxs)
            return vals, idxs

        vals, idxs = lax.fori_loop(
            0, K, round_body,
            (jnp.zeros((8, K), jnp.float32), jnp.zeros((8, K), jnp.int32)))
        _emit(vals, idxs)


def _topk(sim3):
    return pl.pallas_call(
        _topk_body,
        grid=(NQ // 8,),
        in_specs=[pl.BlockSpec((8, NCH, 128), lambda i: (i, 0, 0))],
        out_specs=[
            pl.BlockSpec((8, K), lambda i: (i, 0)),
            pl.BlockSpec((8, K), lambda i: (i, 0)),
        ],
        out_shape=[
            jax.ShapeDtypeStruct((NQ, K), jnp.int32),
            jax.ShapeDtypeStruct((NQ, K), jnp.float32),
        ],
        scratch_shapes=[pltpu.VMEM((8, NCH, 128), jnp.float32)],
    )(sim3)


@functools.lru_cache(maxsize=1)
def _make_sc_vote():
    mesh = plsc.VectorSubcoreMesh(core_axis_name="c", subcore_axis_name="s")

    @functools.partial(
        pl.kernel,
        mesh=mesh,
        compiler_params=pltpu.CompilerParams(needs_layout_passes=False),
        out_type=jax.ShapeDtypeStruct((PATCH * PATCH, NQP), jnp.int32),
        scratch_types=[
            pltpu.VMEM((K, NQP), jnp.int32),
            pltpu.VMEM((K, NQP), jnp.float32),
            pltpu.VMEM((N_TRAIN,), jnp.int32),
            pltpu.VMEM((NUM_CLASSES, 16), jnp.float32),
            pltpu.VMEM((NQP,), jnp.int32),
        ],
    )
    def sc_vote(labels_hbm, idxT_hbm, wT_hbm, out_hbm,
                idx_v, w_v, row_v, hist_v, pred_v):
        wid = lax.axis_index("s") * 2 + lax.axis_index("c")
        pltpu.sync_copy(idxT_hbm, idx_v)
        pltpu.sync_copy(wT_hbm, w_v)
        lanes = lax.iota(jnp.int32, 16)

        def pixel_body(pi, carry):
            p = wid * 8 + pi
            pltpu.sync_copy(labels_hbm.at[p], row_v)

            def pg_body(pg, c2):
                base = pg * 16
                for ci in range(NUM_CLASSES):
                    hist_v[ci, :] = jnp.zeros((16,), jnp.float32)
                for k in range(K):
                    nb = idx_v[k, pl.ds(base, 16)]
                    lbl = plsc.load_gather(row_v, [nb])
                    wt = w_v[k, pl.ds(base, 16)]
                    plsc.addupdate_scatter(hist_v, [lbl, lanes], wt)
                bestv = hist_v[0, :]
                bestc = jnp.zeros((16,), jnp.int32)
                for ci in range(1, NUM_CLASSES):
                    h = hist_v[ci, :]
                    m = h > bestv
                    bestv = jnp.where(m, h, bestv)
                    bestc = jnp.where(m, ci, bestc)
                pred_v[pl.ds(base, 16)] = bestc
                return c2

            lax.fori_loop(0, NQP // 16, pg_body, 0)
            pltpu.sync_copy(pred_v, out_hbm.at[p])
            return carry

        lax.fori_loop(0, (PATCH * PATCH) // 32, pixel_body, 0)

    return sc_vote


def kernel(test_feature, train_features, train_labels):
    q = test_feature.reshape(NQ, D)
    sim = _similarity(q, train_features)
    sim3 = sim.reshape(NQ, NCH, 128)
    idx, w = _topk(sim3)
    idxT = jnp.zeros((K, NQP), jnp.int32).at[:, :NQ].set(idx.T)
    wT = jnp.zeros((K, NQP), jnp.float32).at[:, :NQ].set(w.T)
    pred_pix = _make_sc_vote()(train_labels, idxT, wT)
    pred = (pred_pix[:, :NQ]
            .reshape(PATCH, PATCH, 2, NROWS, NROWS)
            .transpose(2, 3, 0, 4, 1)
            .reshape(2, IMG, IMG))
    return pred
